# Initial kernel scaffold; baseline (speedup 1.0000x reference)
import jax
import jax.numpy as jnp
from jax import lax
from jax.experimental import pallas as pl
from jax.experimental.pallas import tpu as pltpu
from jax.experimental.pallas import tpu_sc as plsc

N = 10000
D = 128
E = 320000
AUG = 144
NC = 2
NS = 16
NW = NC * NS
EPT = E // NW
CH = 80
NCHUNK = EPT // CH
RPT = N // NS


def _stage_a_body(xaug_hbm, src_hbm, dst_hbm, zeros_hbm, out_hbm,
                  agg_sh, src_idx, dst_idx, rows, gsem):
    c = lax.axis_index("c")
    s = lax.axis_index("s")
    base = (c * NS + s) * EPT

    pltpu.sync_copy(zeros_hbm, agg_sh.at[pl.ds(s * RPT, RPT)])
    plsc.subcore_barrier()

    def chunk(i, carry):
        off = base + i * CH
        pltpu.sync_copy(src_hbm.at[pl.ds(off, CH)], src_idx)
        pltpu.sync_copy(dst_hbm.at[pl.ds(off, CH)], dst_idx)
        pltpu.async_copy(xaug_hbm.at[src_idx], rows, gsem).wait()
        pltpu.sync_copy(rows, agg_sh.at[dst_idx], add=True)
        return carry

    lax.fori_loop(0, NCHUNK, chunk, 0)
    plsc.subcore_barrier()

    pltpu.sync_copy(agg_sh.at[pl.ds(s * RPT, RPT)],
                    out_hbm.at[c, pl.ds(s * RPT, RPT)])


def _stage_a(xaug, src, dst, zeros):
    mesh = plsc.VectorSubcoreMesh(core_axis_name="c", subcore_axis_name="s")
    return pl.kernel(
        _stage_a_body,
        out_type=jax.ShapeDtypeStruct((NC, N, AUG), jnp.float32),
        mesh=mesh,
        scratch_types=[
            pltpu.VMEM_SHARED((N, AUG), jnp.float32),
            pltpu.VMEM((CH,), jnp.int32),
            pltpu.VMEM((CH,), jnp.int32),
            pltpu.VMEM((CH, AUG), jnp.float32),
            pltpu.SemaphoreType.DMA,
        ],
    )(xaug, src, dst, zeros)


def _stage_b_kernel(p0_ref, p1_ref, wg_ref, bg_ref, w1_ref, b1_ref,
                    w2_ref, b2_ref, out_ref):
    agg = p0_ref[...] + p1_ref[...]
    deg = jnp.maximum(agg[:, D:D + 1], 1.0)
    mean = agg[:, :D] / deg
    xe = jnp.maximum(
        jnp.dot(mean, wg_ref[...], preferred_element_type=jnp.float32)
        + bg_ref[...], 0.0)
    h = jnp.maximum(
        jnp.dot(xe, w1_ref[...], preferred_element_type=jnp.float32)
        + b1_ref[...], 0.0)
    phi = jnp.dot(h, w2_ref[...], preferred_element_type=jnp.float32) + b2_ref[...]
    rows = xe.shape[0]
    out_ref[...] = jnp.concatenate(
        [xe[:, 0:3], phi, jnp.zeros((rows, 4), jnp.float32)], axis=1)


def _stage_b(p0, p1, wg, bg, w1, b1, w2, b2):
    blk = 1000
    grid = N // blk
    return pl.pallas_call(
        _stage_b_kernel,
        grid=(grid,),
        in_specs=[
            pl.BlockSpec((blk, AUG), lambda i: (i, 0)),
            pl.BlockSpec((blk, AUG), lambda i: (i, 0)),
            pl.BlockSpec((D, D), lambda i: (0, 0)),
            pl.BlockSpec((1, D), lambda i: (0, 0)),
            pl.BlockSpec((D, 64), lambda i: (0, 0)),
            pl.BlockSpec((1, 64), lambda i: (0, 0)),
            pl.BlockSpec((64, 1), lambda i: (0, 0)),
            pl.BlockSpec((1, 1), lambda i: (0, 0)),
        ],
        out_specs=pl.BlockSpec((blk, 8), lambda i: (i, 0)),
        out_shape=jax.ShapeDtypeStruct((N, 8), jnp.float32),
    )(p0, p1, wg, bg, w1, b1, w2, b2)


def _stage_c_body(small_hbm, src_hbm, dst_hbm, out_hbm,
                  si, di, ei, ej, fbig, sem1, sem2):
    c = lax.axis_index("c")
    s = lax.axis_index("s")
    wid = c * NS + s
    base = wid * EPT
    col = [jnp.full((16,), k, jnp.int32) for k in range(4)]

    def chunk(i, carry):
        off = base + i * CH
        pltpu.sync_copy(src_hbm.at[pl.ds(off, CH)], si)
        pltpu.sync_copy(dst_hbm.at[pl.ds(off, CH)], di)
        d1 = pltpu.async_copy(small_hbm.at[si], ei, sem1)
        d2 = pltpu.async_copy(small_hbm.at[di], ej, sem2)
        d1.wait()
        d2.wait()
        for v in range(CH // 16):
            r16 = lax.iota(jnp.int32, 16) + v * 16
            ei0 = plsc.load_gather(ei, [r16, col[0]])
            ei1 = plsc.load_gather(ei, [r16, col[1]])
            ei2 = plsc.load_gather(ei, [r16, col[2]])
            phi = plsc.load_gather(ei, [r16, col[3]])
            ej0 = plsc.load_gather(ej, [r16, col[0]])
            ej1 = plsc.load_gather(ej, [r16, col[1]])
            ej2 = plsc.load_gather(ej, [r16, col[2]])
            o = i * CH + v * 16
            fbig[0, pl.ds(o, 16)] = ej1 - ei1
            fbig[1, pl.ds(o, 16)] = ej2 - ei2
            fbig[2, pl.ds(o, 16)] = ej0 - ei0
            fbig[3, pl.ds(o, 16)] = phi
        return carry

    lax.fori_loop(0, NCHUNK, chunk, 0)
    pltpu.sync_copy(fbig, out_hbm.at[wid])


def _stage_c(small, src, dst):
    mesh = plsc.VectorSubcoreMesh(core_axis_name="c", subcore_axis_name="s")
    return pl.kernel(
        _stage_c_body,
        out_type=jax.ShapeDtypeStruct((NW, 4, EPT), jnp.float32),
        mesh=mesh,
        scratch_types=[
            pltpu.VMEM((CH,), jnp.int32),
            pltpu.VMEM((CH,), jnp.int32),
            pltpu.VMEM((CH, 8), jnp.float32),
            pltpu.VMEM((CH, 8), jnp.float32),
            pltpu.VMEM((4, EPT), jnp.float32),
            pltpu.SemaphoreType.DMA,
            pltpu.SemaphoreType.DMA,
        ],
    )(small, src, dst)


def _stage_d_kernel(f_ref, w1t_ref, b1_ref, w2t_ref, b2_ref, out_ref):
    f = f_ref[...]
    dx = f[0:1, :]
    dy = f[1:2, :]
    dt = f[2:3, :]
    phi = f[3:4, :]
    dx2 = dx * dx + dy * dy
    ds2 = dx2 - phi * (dt * dt)
    sp = jnp.sqrt(dx2)
    itl = jnp.where(ds2 < 0, 1.0, 0.0).astype(jnp.float32)
    zz = jnp.zeros_like(dx)
    feats = jnp.concatenate([dx, dy, dt, ds2, sp, itl, zz, zz], axis=0)
    h = jnp.maximum(
        jnp.dot(w1t_ref[...], feats, preferred_element_type=jnp.float32)
        + b1_ref[...], 0.0)
    o = jnp.dot(w2t_ref[...], h, preferred_element_type=jnp.float32) + b2_ref[...]
    out_ref[...] = jax.nn.sigmoid(o)


def _stage_d(feat2d, w1t, b1c, w2t, b2):
    return pl.pallas_call(
        _stage_d_kernel,
        grid=(NW,),
        in_specs=[
            pl.BlockSpec((4, EPT), lambda i: (i, 0)),
            pl.BlockSpec((32, 8), lambda i: (0, 0)),
            pl.BlockSpec((32, 1), lambda i: (0, 0)),
            pl.BlockSpec((1, 32), lambda i: (0, 0)),
            pl.BlockSpec((1, 1), lambda i: (0, 0)),
        ],
        out_specs=pl.BlockSpec((1, EPT), lambda i: (i, 0)),
        out_shape=jax.ShapeDtypeStruct((NW, EPT), jnp.float32),
    )(feat2d, w1t, b1c, w2t, b2)


def kernel(x, edge_index, W_gnn, b_gnn, W_phi1, b_phi1, W_phi2, b_phi2,
           W_ep1, b_ep1, W_ep2, b_ep2):
    src = edge_index[0]
    dst = edge_index[1]
    xaug = jnp.concatenate(
        [x, jnp.ones((N, 1), jnp.float32), jnp.zeros((N, AUG - D - 1), jnp.float32)],
        axis=1)
    zeros = jnp.zeros((RPT, AUG), jnp.float32)

    partials = _stage_a(xaug, src, dst, zeros)
    small = _stage_b(partials[0], partials[1], W_gnn, b_gnn.reshape(1, D),
                     W_phi1, b_phi1.reshape(1, 64), W_phi2, b_phi2.reshape(1, 1))
    feat = _stage_c(small, src, dst)

    w1t = jnp.concatenate([W_ep1, jnp.zeros((2, 32), jnp.float32)], axis=0).T
    probs = _stage_d(feat.reshape(NW * 4, EPT), w1t, b_ep1.reshape(32, 1),
                     W_ep2.T, b_ep2.reshape(1, 1))
    return probs.reshape(E)


# speedup vs baseline: 5.8104x; 5.8104x over previous
import jax
import jax.numpy as jnp
from jax import lax
from jax.experimental import pallas as pl
from jax.experimental.pallas import tpu as pltpu
from jax.experimental.pallas import tpu_sc as plsc

N = 10000
D = 128
E = 320000
AUG = 144
NC = 2
NS = 16
NW = NC * NS
EPT = E // NW
CH = 80
NCHUNK = EPT // CH
RPT = 624
TAIL = N - NS * RPT


def _stage_a_body(xaug_hbm, src_hbm, dst_hbm, zeros_hbm, out_hbm,
                  agg_sh, src_idx, dst_idx, rows, gsem):
    c = lax.axis_index("c")
    s = lax.axis_index("s")
    base = (c * NS + s) * EPT

    pltpu.sync_copy(zeros_hbm.at[pl.ds(0, RPT)], agg_sh.at[pl.ds(s * RPT, RPT)])

    @pl.when(s == NS - 1)
    def _():
        pltpu.sync_copy(zeros_hbm.at[pl.ds(0, TAIL)],
                        agg_sh.at[pl.ds(NS * RPT, TAIL)])

    plsc.subcore_barrier()

    def chunk(i, carry):
        off = base + i * CH
        pltpu.sync_copy(src_hbm.at[pl.ds(off, CH)], src_idx)
        pltpu.sync_copy(dst_hbm.at[pl.ds(off, CH)], dst_idx)
        pltpu.async_copy(xaug_hbm.at[src_idx], rows, gsem).wait()
        pltpu.sync_copy(rows, agg_sh.at[dst_idx], add=True)
        return carry

    lax.fori_loop(0, NCHUNK, chunk, 0)
    plsc.subcore_barrier()

    pltpu.sync_copy(agg_sh.at[pl.ds(s * RPT, RPT)],
                    out_hbm.at[c, pl.ds(s * RPT, RPT)])

    @pl.when(s == NS - 1)
    def _():
        pltpu.sync_copy(agg_sh.at[pl.ds(NS * RPT, TAIL)],
                        out_hbm.at[c, pl.ds(NS * RPT, TAIL)])


def _stage_a(xaug, src, dst, zeros):
    mesh = plsc.VectorSubcoreMesh(core_axis_name="c", subcore_axis_name="s")
    return pl.kernel(
        _stage_a_body,
        out_type=jax.ShapeDtypeStruct((NC, N, AUG), jnp.float32),
        mesh=mesh,
        scratch_types=[
            pltpu.VMEM_SHARED((N, AUG), jnp.float32),
            pltpu.VMEM((CH,), jnp.int32),
            pltpu.VMEM((CH,), jnp.int32),
            pltpu.VMEM((CH, AUG), jnp.float32),
            pltpu.SemaphoreType.DMA,
        ],
        compiler_params=pltpu.CompilerParams(use_tc_tiling_on_sc=False, needs_layout_passes=False),
    )(xaug, src, dst, zeros)


def _stage_b_kernel(p0_ref, p1_ref, wg_ref, bg_ref, w1_ref, b1_ref,
                    w2_ref, b2_ref, out_ref):
    agg = p0_ref[...] + p1_ref[...]
    deg = jnp.maximum(agg[:, D:D + 1], 1.0)
    mean = agg[:, :D] / deg
    xe = jnp.maximum(
        jnp.dot(mean, wg_ref[...], preferred_element_type=jnp.float32)
        + bg_ref[...], 0.0)
    h = jnp.maximum(
        jnp.dot(xe, w1_ref[...], preferred_element_type=jnp.float32)
        + b1_ref[...], 0.0)
    phi = jnp.dot(h, w2_ref[...], preferred_element_type=jnp.float32) + b2_ref[...]
    rows = xe.shape[0]
    out_ref[...] = jnp.concatenate(
        [xe[:, 0:3], phi, jnp.zeros((rows, 4), jnp.float32)], axis=1)


def _stage_b(p0, p1, wg, bg, w1, b1, w2, b2):
    blk = 1000
    grid = N // blk
    return pl.pallas_call(
        _stage_b_kernel,
        grid=(grid,),
        in_specs=[
            pl.BlockSpec((blk, AUG), lambda i: (i, 0)),
            pl.BlockSpec((blk, AUG), lambda i: (i, 0)),
            pl.BlockSpec((D, D), lambda i: (0, 0)),
            pl.BlockSpec((1, D), lambda i: (0, 0)),
            pl.BlockSpec((D, 64), lambda i: (0, 0)),
            pl.BlockSpec((1, 64), lambda i: (0, 0)),
            pl.BlockSpec((64, 1), lambda i: (0, 0)),
            pl.BlockSpec((1, 1), lambda i: (0, 0)),
        ],
        out_specs=pl.BlockSpec((blk, 8), lambda i: (i, 0)),
        out_shape=jax.ShapeDtypeStruct((N, 8), jnp.float32),
    )(p0, p1, wg, bg, w1, b1, w2, b2)


def _stage_c_body(small_hbm, src_hbm, dst_hbm, out_hbm,
                  si, di, ei, ej, fbig, sem1, sem2):
    c = lax.axis_index("c")
    s = lax.axis_index("s")
    wid = c * NS + s
    base = wid * EPT
    col = [jnp.full((16,), k, jnp.int32) for k in range(4)]

    def chunk(i, carry):
        off = base + i * CH
        pltpu.sync_copy(src_hbm.at[pl.ds(off, CH)], si)
        pltpu.sync_copy(dst_hbm.at[pl.ds(off, CH)], di)
        d1 = pltpu.async_copy(small_hbm.at[si], ei, sem1)
        d2 = pltpu.async_copy(small_hbm.at[di], ej, sem2)
        d1.wait()
        d2.wait()
        for v in range(CH // 16):
            r16 = lax.iota(jnp.int32, 16) + v * 16
            ei0 = plsc.load_gather(ei, [r16, col[0]])
            ei1 = plsc.load_gather(ei, [r16, col[1]])
            ei2 = plsc.load_gather(ei, [r16, col[2]])
            phi = plsc.load_gather(ei, [r16, col[3]])
            ej0 = plsc.load_gather(ej, [r16, col[0]])
            ej1 = plsc.load_gather(ej, [r16, col[1]])
            ej2 = plsc.load_gather(ej, [r16, col[2]])
            o = i * CH + v * 16
            fbig[0, pl.ds(o, 16)] = ej1 - ei1
            fbig[1, pl.ds(o, 16)] = ej2 - ei2
            fbig[2, pl.ds(o, 16)] = ej0 - ei0
            fbig[3, pl.ds(o, 16)] = phi
        return carry

    lax.fori_loop(0, NCHUNK, chunk, 0)
    pltpu.sync_copy(fbig, out_hbm.at[wid])


def _stage_c(small, src, dst):
    mesh = plsc.VectorSubcoreMesh(core_axis_name="c", subcore_axis_name="s")
    return pl.kernel(
        _stage_c_body,
        out_type=jax.ShapeDtypeStruct((NW, 4, EPT), jnp.float32),
        mesh=mesh,
        scratch_types=[
            pltpu.VMEM((CH,), jnp.int32),
            pltpu.VMEM((CH,), jnp.int32),
            pltpu.VMEM((CH, 8), jnp.float32),
            pltpu.VMEM((CH, 8), jnp.float32),
            pltpu.VMEM((4, EPT), jnp.float32),
            pltpu.SemaphoreType.DMA,
            pltpu.SemaphoreType.DMA,
        ],
        compiler_params=pltpu.CompilerParams(use_tc_tiling_on_sc=False, needs_layout_passes=False),
    )(small, src, dst)


def _stage_d_kernel(f_ref, w1t_ref, b1_ref, w2t_ref, b2_ref, out_ref):
    f = f_ref[0]
    dx = f[0:1, :]
    dy = f[1:2, :]
    dt = f[2:3, :]
    phi = f[3:4, :]
    dx2 = dx * dx + dy * dy
    ds2 = dx2 - phi * (dt * dt)
    sp = jnp.sqrt(dx2)
    itl = jnp.where(ds2 < 0, 1.0, 0.0).astype(jnp.float32)
    zz = jnp.zeros_like(dx)
    feats = jnp.concatenate([dx, dy, dt, ds2, sp, itl, zz, zz], axis=0)
    h = jnp.maximum(
        jnp.dot(w1t_ref[...], feats, preferred_element_type=jnp.float32)
        + b1_ref[...], 0.0)
    o = jnp.dot(w2t_ref[...], h, preferred_element_type=jnp.float32) + b2_ref[...]
    out_ref[...] = jax.nn.sigmoid(o)[None]


def _stage_d(feat3d, w1t, b1c, w2t, b2):
    return pl.pallas_call(
        _stage_d_kernel,
        grid=(NW,),
        in_specs=[
            pl.BlockSpec((1, 4, EPT), lambda i: (i, 0, 0)),
            pl.BlockSpec((32, 8), lambda i: (0, 0)),
            pl.BlockSpec((32, 1), lambda i: (0, 0)),
            pl.BlockSpec((1, 32), lambda i: (0, 0)),
            pl.BlockSpec((1, 1), lambda i: (0, 0)),
        ],
        out_specs=pl.BlockSpec((1, 1, EPT), lambda i: (i, 0, 0)),
        out_shape=jax.ShapeDtypeStruct((NW, 1, EPT), jnp.float32),
    )(feat3d, w1t, b1c, w2t, b2)


def kernel(x, edge_index, W_gnn, b_gnn, W_phi1, b_phi1, W_phi2, b_phi2,
           W_ep1, b_ep1, W_ep2, b_ep2):
    src = edge_index[0]
    dst = edge_index[1]
    xaug = jnp.concatenate(
        [x, jnp.ones((N, 1), jnp.float32), jnp.zeros((N, AUG - D - 1), jnp.float32)],
        axis=1)
    zeros = jnp.zeros((RPT, AUG), jnp.float32)

    partials = _stage_a(xaug, src, dst, zeros)
    small = _stage_b(partials[0], partials[1], W_gnn, b_gnn.reshape(1, D),
                     W_phi1, b_phi1.reshape(1, 64), W_phi2, b_phi2.reshape(1, 1))
    feat = _stage_c(small, src, dst)

    w1t = jnp.concatenate([W_ep1, jnp.zeros((2, 32), jnp.float32)], axis=0).T
    probs = _stage_d(feat, w1t, b_ep1.reshape(32, 1),
                     W_ep2.T, b_ep2.reshape(1, 1))
    return probs.reshape(E)


# speedup vs baseline: 12.8722x; 2.2154x over previous
import jax
import jax.numpy as jnp
from jax import lax
from jax.experimental import pallas as pl
from jax.experimental.pallas import tpu as pltpu
from jax.experimental.pallas import tpu_sc as plsc

N = 10000
D = 128
E = 320000
HALF = 64
AUGH = 80
NC = 2
NS = 16
NW = NC * NS
EPT = E // NW
EPS = E // NS
CH = 80
NCHUNK = EPT // CH
NCHA = EPS // CH
RPT = 624
TAIL = N - NS * RPT
KBUF = 5


def _stage_a_body(xcat_hbm, srca_hbm, dsta_hbm, zeros_hbm, out_hbm,
                  agg_sh, src2d, dst2d,
                  r0, r1, r2, r3, r4, g0, g1, g2, g3, g4):
    c = lax.axis_index("c")
    s = lax.axis_index("s")
    rows = [r0, r1, r2, r3, r4]
    gsem = [g0, g1, g2, g3, g4]

    pltpu.sync_copy(srca_hbm.at[c, s], src2d)
    pltpu.sync_copy(dsta_hbm.at[s], dst2d)
    for b in range(KBUF):
        pltpu.async_copy(xcat_hbm.at[src2d.at[b]], rows[b], gsem[b])

    pltpu.sync_copy(zeros_hbm.at[pl.ds(0, RPT)], agg_sh.at[pl.ds(s * RPT, RPT)])

    @pl.when(s == NS - 1)
    def _():
        pltpu.sync_copy(zeros_hbm.at[pl.ds(0, TAIL)],
                        agg_sh.at[pl.ds(NS * RPT, TAIL)])

    plsc.subcore_barrier()

    def outer(o, carry):
        for b in range(KBUF):
            g = o * KBUF + b
            pltpu.make_async_copy(xcat_hbm.at[pl.ds(0, CH)], rows[b],
                                  gsem[b]).wait()
            pltpu.sync_copy(rows[b], agg_sh.at[dst2d.at[g]], add=True)
            nxt = g + KBUF

            @pl.when(nxt < NCHA)
            def _():
                pltpu.async_copy(xcat_hbm.at[src2d.at[nxt]], rows[b], gsem[b])
        return carry

    lax.fori_loop(0, NCHA // KBUF, outer, 0)
    plsc.subcore_barrier()

    pltpu.sync_copy(agg_sh.at[pl.ds(s * RPT, RPT)],
                    out_hbm.at[c, pl.ds(s * RPT, RPT)])

    @pl.when(s == NS - 1)
    def _():
        pltpu.sync_copy(agg_sh.at[pl.ds(NS * RPT, TAIL)],
                        out_hbm.at[c, pl.ds(NS * RPT, TAIL)])


def _stage_a(xcat, srca, dsta, zeros):
    mesh = plsc.VectorSubcoreMesh(core_axis_name="c", subcore_axis_name="s")
    return pl.kernel(
        _stage_a_body,
        out_type=jax.ShapeDtypeStruct((NC, N, AUGH), jnp.float32),
        mesh=mesh,
        scratch_types=[
            pltpu.VMEM_SHARED((N, AUGH), jnp.float32),
            pltpu.VMEM((NCHA, CH), jnp.int32),
            pltpu.VMEM((NCHA, CH), jnp.int32),
        ] + [pltpu.VMEM((CH, AUGH), jnp.float32)] * KBUF
          + [pltpu.SemaphoreType.DMA] * KBUF,
        compiler_params=pltpu.CompilerParams(use_tc_tiling_on_sc=False, needs_layout_passes=False),
    )(xcat, srca, dsta, zeros)


def _stage_b_kernel(p0_ref, p1_ref, wg_ref, bg_ref, w1_ref, b1_ref,
                    w2_ref, b2_ref, out_ref):
    p0 = p0_ref[...]
    p1 = p1_ref[...]
    deg = jnp.maximum(p0[:, HALF:HALF + 1], 1.0)
    mean = jnp.concatenate([p0[:, :HALF], p1[:, :HALF]], axis=1) / deg
    xe = jnp.maximum(
        jnp.dot(mean, wg_ref[...], preferred_element_type=jnp.float32)
        + bg_ref[...], 0.0)
    h = jnp.maximum(
        jnp.dot(xe, w1_ref[...], preferred_element_type=jnp.float32)
        + b1_ref[...], 0.0)
    phi = jnp.dot(h, w2_ref[...], preferred_element_type=jnp.float32) + b2_ref[...]
    rows = xe.shape[0]
    out_ref[...] = jnp.concatenate(
        [xe[:, 0:3], phi, jnp.zeros((rows, 4), jnp.float32)], axis=1)


def _stage_b(p0, p1, wg, bg, w1, b1, w2, b2):
    blk = 1000
    grid = N // blk
    return pl.pallas_call(
        _stage_b_kernel,
        grid=(grid,),
        in_specs=[
            pl.BlockSpec((blk, AUGH), lambda i: (i, 0)),
            pl.BlockSpec((blk, AUGH), lambda i: (i, 0)),
            pl.BlockSpec((D, D), lambda i: (0, 0)),
            pl.BlockSpec((1, D), lambda i: (0, 0)),
            pl.BlockSpec((D, 64), lambda i: (0, 0)),
            pl.BlockSpec((1, 64), lambda i: (0, 0)),
            pl.BlockSpec((64, 1), lambda i: (0, 0)),
            pl.BlockSpec((1, 1), lambda i: (0, 0)),
        ],
        out_specs=pl.BlockSpec((blk, 8), lambda i: (i, 0)),
        out_shape=jax.ShapeDtypeStruct((N, 8), jnp.float32),
    )(p0, p1, wg, bg, w1, b1, w2, b2)


def _stage_c_body(small_hbm, src3_hbm, dst3_hbm, out_hbm,
                  src2d, dst2d, fbig,
                  ei0b, ei1b, ei2b, ei3b, ei4b,
                  ej0b, ej1b, ej2b, ej3b, ej4b,
                  gi0, gi1, gi2, gi3, gi4,
                  gj0, gj1, gj2, gj3, gj4):
    c = lax.axis_index("c")
    s = lax.axis_index("s")
    wid = c * NS + s
    eib = [ei0b, ei1b, ei2b, ei3b, ei4b]
    ejb = [ej0b, ej1b, ej2b, ej3b, ej4b]
    gi = [gi0, gi1, gi2, gi3, gi4]
    gj = [gj0, gj1, gj2, gj3, gj4]
    col = [jnp.full((16,), k, jnp.int32) for k in range(4)]

    pltpu.sync_copy(src3_hbm.at[wid], src2d)
    pltpu.sync_copy(dst3_hbm.at[wid], dst2d)
    for b in range(KBUF):
        pltpu.async_copy(small_hbm.at[src2d.at[b]], eib[b], gi[b])
        pltpu.async_copy(small_hbm.at[dst2d.at[b]], ejb[b], gj[b])

    def outer(o, carry):
        for b in range(KBUF):
            g = o * KBUF + b
            pltpu.make_async_copy(small_hbm.at[pl.ds(0, CH)], eib[b],
                                  gi[b]).wait()
            pltpu.make_async_copy(small_hbm.at[pl.ds(0, CH)], ejb[b],
                                  gj[b]).wait()
            for v in range(CH // 16):
                r16 = lax.iota(jnp.int32, 16) + v * 16
                ei0 = plsc.load_gather(eib[b], [r16, col[0]])
                ei1 = plsc.load_gather(eib[b], [r16, col[1]])
                ei2 = plsc.load_gather(eib[b], [r16, col[2]])
                phi = plsc.load_gather(eib[b], [r16, col[3]])
                ej0 = plsc.load_gather(ejb[b], [r16, col[0]])
                ej1 = plsc.load_gather(ejb[b], [r16, col[1]])
                ej2 = plsc.load_gather(ejb[b], [r16, col[2]])
                o16 = g * CH + v * 16
                fbig[0, pl.ds(o16, 16)] = ej1 - ei1
                fbig[1, pl.ds(o16, 16)] = ej2 - ei2
                fbig[2, pl.ds(o16, 16)] = ej0 - ei0
                fbig[3, pl.ds(o16, 16)] = phi
            nxt = g + KBUF

            @pl.when(nxt < NCHUNK)
            def _():
                pltpu.async_copy(small_hbm.at[src2d.at[nxt]], eib[b], gi[b])
                pltpu.async_copy(small_hbm.at[dst2d.at[nxt]], ejb[b], gj[b])
        return carry

    lax.fori_loop(0, NCHUNK // KBUF, outer, 0)
    pltpu.sync_copy(fbig, out_hbm.at[wid])


def _stage_c(small, src3, dst3):
    mesh = plsc.VectorSubcoreMesh(core_axis_name="c", subcore_axis_name="s")
    return pl.kernel(
        _stage_c_body,
        out_type=jax.ShapeDtypeStruct((NW, 4, EPT), jnp.float32),
        mesh=mesh,
        scratch_types=[
            pltpu.VMEM((NCHUNK, CH), jnp.int32),
            pltpu.VMEM((NCHUNK, CH), jnp.int32),
            pltpu.VMEM((4, EPT), jnp.float32),
        ] + [pltpu.VMEM((CH, 8), jnp.float32)] * (2 * KBUF)
          + [pltpu.SemaphoreType.DMA] * (2 * KBUF),
        compiler_params=pltpu.CompilerParams(use_tc_tiling_on_sc=False, needs_layout_passes=False),
    )(small, src3, dst3)


def _stage_d_kernel(f_ref, w1t_ref, b1_ref, w2t_ref, b2_ref, out_ref):
    f = f_ref[0]
    dx = f[0:1, :]
    dy = f[1:2, :]
    dt = f[2:3, :]
    phi = f[3:4, :]
    dx2 = dx * dx + dy * dy
    ds2 = dx2 - phi * (dt * dt)
    sp = jnp.sqrt(dx2)
    itl = jnp.where(ds2 < 0, 1.0, 0.0).astype(jnp.float32)
    zz = jnp.zeros_like(dx)
    feats = jnp.concatenate([dx, dy, dt, ds2, sp, itl, zz, zz], axis=0)
    h = jnp.maximum(
        jnp.dot(w1t_ref[...], feats, preferred_element_type=jnp.float32)
        + b1_ref[...], 0.0)
    o = jnp.dot(w2t_ref[...], h, preferred_element_type=jnp.float32) + b2_ref[...]
    out_ref[...] = jax.nn.sigmoid(o)[None]


def _stage_d(feat3d, w1t, b1c, w2t, b2):
    return pl.pallas_call(
        _stage_d_kernel,
        grid=(NW,),
        in_specs=[
            pl.BlockSpec((1, 4, EPT), lambda i: (i, 0, 0)),
            pl.BlockSpec((32, 8), lambda i: (0, 0)),
            pl.BlockSpec((32, 1), lambda i: (0, 0)),
            pl.BlockSpec((1, 32), lambda i: (0, 0)),
            pl.BlockSpec((1, 1), lambda i: (0, 0)),
        ],
        out_specs=pl.BlockSpec((1, 1, EPT), lambda i: (i, 0, 0)),
        out_shape=jax.ShapeDtypeStruct((NW, 1, EPT), jnp.float32),
    )(feat3d, w1t, b1c, w2t, b2)


def kernel(x, edge_index, W_gnn, b_gnn, W_phi1, b_phi1, W_phi2, b_phi2,
           W_ep1, b_ep1, W_ep2, b_ep2):
    src3 = edge_index[0].reshape(NW, NCHUNK, CH)
    dst3 = edge_index[1].reshape(NW, NCHUNK, CH)
    ones = jnp.ones((N, 1), jnp.float32)
    padz = jnp.zeros((N, AUGH - HALF - 1), jnp.float32)
    xcat = jnp.concatenate(
        [jnp.concatenate([x[:, :HALF], ones, padz], axis=1),
         jnp.concatenate([x[:, HALF:], ones, padz], axis=1)], axis=0)
    srcr = edge_index[0].reshape(NS, NCHA, CH)
    srca = jnp.stack([srcr, srcr + N])
    dsta = edge_index[1].reshape(NS, NCHA, CH)
    zeros = jnp.zeros((RPT, AUGH), jnp.float32)

    partials = _stage_a(xcat, srca, dsta, zeros)
    small = _stage_b(partials[0], partials[1], W_gnn, b_gnn.reshape(1, D),
                     W_phi1, b_phi1.reshape(1, 64), W_phi2, b_phi2.reshape(1, 1))
    feat = _stage_c(small, src3, dst3)

    w1t = jnp.concatenate([W_ep1, jnp.zeros((2, 32), jnp.float32)], axis=0).T
    probs = _stage_d(feat, w1t, b_ep1.reshape(32, 1),
                     W_ep2.T, b_ep2.reshape(1, 1))
    return probs.reshape(E)


# speedup vs baseline: 13.6099x; 1.0573x over previous
import jax
import jax.numpy as jnp
from jax import lax
from jax.experimental import pallas as pl
from jax.experimental.pallas import tpu as pltpu
from jax.experimental.pallas import tpu_sc as plsc

N = 10000
D = 128
E = 320000
HALF = 64
AUGH = 80
NC = 2
NS = 16
NW = NC * NS
EPT = E // NW
EPS = E // NS
CH = 80
NCHUNK = EPT // CH
NCHA = EPS // CH
RPT = 624
TAIL = N - NS * RPT
KBUF = 5


def _stage_a_body(xcat_hbm, eidx_hbm, zeros_hbm, out_hbm,
                  agg_sh, src2d, dst2d,
                  r0, r1, r2, r3, r4, g0, g1, g2, g3, g4):
    c = lax.axis_index("c")
    s = lax.axis_index("s")
    rows = [r0, r1, r2, r3, r4]
    gsem = [g0, g1, g2, g3, g4]

    pltpu.sync_copy(eidx_hbm.at[0, pl.ds(s * NCHA, NCHA)], src2d)
    pltpu.sync_copy(eidx_hbm.at[1, pl.ds(s * NCHA, NCHA)], dst2d)
    bias = jnp.broadcast_to(c * N, (16,)).astype(jnp.int32)

    def add_bias(r, carry):
        for k in range(CH // 16):
            src2d[r, pl.ds(k * 16, 16)] = src2d[r, pl.ds(k * 16, 16)] + bias
        return carry

    lax.fori_loop(0, NCHA, add_bias, 0)
    for b in range(KBUF):
        pltpu.async_copy(xcat_hbm.at[src2d.at[b]], rows[b], gsem[b])

    pltpu.sync_copy(zeros_hbm.at[pl.ds(0, RPT)], agg_sh.at[pl.ds(s * RPT, RPT)])

    @pl.when(s == NS - 1)
    def _():
        pltpu.sync_copy(zeros_hbm.at[pl.ds(0, TAIL)],
                        agg_sh.at[pl.ds(NS * RPT, TAIL)])

    plsc.subcore_barrier()

    def outer(o, carry):
        for b in range(KBUF):
            g = o * KBUF + b
            pltpu.make_async_copy(xcat_hbm.at[pl.ds(0, CH)], rows[b],
                                  gsem[b]).wait()
            pltpu.sync_copy(rows[b], agg_sh.at[dst2d.at[g]], add=True)
            nxt = g + KBUF

            @pl.when(nxt < NCHA)
            def _():
                pltpu.async_copy(xcat_hbm.at[src2d.at[nxt]], rows[b], gsem[b])
        return carry

    lax.fori_loop(0, NCHA // KBUF, outer, 0)
    plsc.subcore_barrier()

    pltpu.sync_copy(agg_sh.at[pl.ds(s * RPT, RPT)],
                    out_hbm.at[c, pl.ds(s * RPT, RPT)])

    @pl.when(s == NS - 1)
    def _():
        pltpu.sync_copy(agg_sh.at[pl.ds(NS * RPT, TAIL)],
                        out_hbm.at[c, pl.ds(NS * RPT, TAIL)])


def _stage_a(xcat, eidx3, zeros):
    mesh = plsc.VectorSubcoreMesh(core_axis_name="c", subcore_axis_name="s")
    return pl.kernel(
        _stage_a_body,
        out_type=jax.ShapeDtypeStruct((NC, N, AUGH), jnp.float32),
        mesh=mesh,
        scratch_types=[
            pltpu.VMEM_SHARED((N, AUGH), jnp.float32),
            pltpu.VMEM((NCHA, CH), jnp.int32),
            pltpu.VMEM((NCHA, CH), jnp.int32),
        ] + [pltpu.VMEM((CH, AUGH), jnp.float32)] * KBUF
          + [pltpu.SemaphoreType.DMA] * KBUF,
        compiler_params=pltpu.CompilerParams(use_tc_tiling_on_sc=False, needs_layout_passes=False),
    )(xcat, eidx3, zeros)


def _stage_b_kernel(p0_ref, p1_ref, wg_ref, bg_ref, w1_ref, b1_ref,
                    w2_ref, b2_ref, out_ref):
    p0 = p0_ref[...]
    p1 = p1_ref[...]
    deg = jnp.maximum(p0[:, HALF:HALF + 1], 1.0)
    mean = jnp.concatenate([p0[:, :HALF], p1[:, :HALF]], axis=1) / deg
    xe = jnp.maximum(
        jnp.dot(mean, wg_ref[...], preferred_element_type=jnp.float32)
        + bg_ref[...], 0.0)
    h = jnp.maximum(
        jnp.dot(xe, w1_ref[...], preferred_element_type=jnp.float32)
        + b1_ref[...], 0.0)
    phi = jnp.dot(h, w2_ref[...], preferred_element_type=jnp.float32) + b2_ref[...]
    rows = xe.shape[0]
    out_ref[...] = jnp.concatenate(
        [xe[:, 0:3], phi, jnp.zeros((rows, 4), jnp.float32)], axis=1)


def _stage_b(p0, p1, wg, bg, w1, b1, w2, b2):
    blk = 1000
    grid = N // blk
    return pl.pallas_call(
        _stage_b_kernel,
        grid=(grid,),
        in_specs=[
            pl.BlockSpec((blk, AUGH), lambda i: (i, 0)),
            pl.BlockSpec((blk, AUGH), lambda i: (i, 0)),
            pl.BlockSpec((D, D), lambda i: (0, 0)),
            pl.BlockSpec((1, D), lambda i: (0, 0)),
            pl.BlockSpec((D, 64), lambda i: (0, 0)),
            pl.BlockSpec((1, 64), lambda i: (0, 0)),
            pl.BlockSpec((64, 1), lambda i: (0, 0)),
            pl.BlockSpec((1, 1), lambda i: (0, 0)),
        ],
        out_specs=pl.BlockSpec((blk, 8), lambda i: (i, 0)),
        out_shape=jax.ShapeDtypeStruct((N, 8), jnp.float32),
    )(p0, p1, wg, bg, w1, b1, w2, b2)


def _stage_c_body(small_hbm, eidx_hbm, out_hbm,
                  src2d, dst2d, fbig,
                  ei0b, ei1b, ei2b, ei3b, ei4b,
                  ej0b, ej1b, ej2b, ej3b, ej4b,
                  gi0, gi1, gi2, gi3, gi4,
                  gj0, gj1, gj2, gj3, gj4):
    c = lax.axis_index("c")
    s = lax.axis_index("s")
    wid = c * NS + s
    eib = [ei0b, ei1b, ei2b, ei3b, ei4b]
    ejb = [ej0b, ej1b, ej2b, ej3b, ej4b]
    gi = [gi0, gi1, gi2, gi3, gi4]
    gj = [gj0, gj1, gj2, gj3, gj4]
    col = [jnp.full((16,), k, jnp.int32) for k in range(4)]

    pltpu.sync_copy(eidx_hbm.at[0, pl.ds(wid * NCHUNK, NCHUNK)], src2d)
    pltpu.sync_copy(eidx_hbm.at[1, pl.ds(wid * NCHUNK, NCHUNK)], dst2d)
    for b in range(KBUF):
        pltpu.async_copy(small_hbm.at[src2d.at[b]], eib[b], gi[b])
        pltpu.async_copy(small_hbm.at[dst2d.at[b]], ejb[b], gj[b])

    def outer(o, carry):
        for b in range(KBUF):
            g = o * KBUF + b
            pltpu.make_async_copy(small_hbm.at[pl.ds(0, CH)], eib[b],
                                  gi[b]).wait()
            pltpu.make_async_copy(small_hbm.at[pl.ds(0, CH)], ejb[b],
                                  gj[b]).wait()
            for v in range(CH // 16):
                r16 = lax.iota(jnp.int32, 16) + v * 16
                ei0 = plsc.load_gather(eib[b], [r16, col[0]])
                ei1 = plsc.load_gather(eib[b], [r16, col[1]])
                ei2 = plsc.load_gather(eib[b], [r16, col[2]])
                phi = plsc.load_gather(eib[b], [r16, col[3]])
                ej0 = plsc.load_gather(ejb[b], [r16, col[0]])
                ej1 = plsc.load_gather(ejb[b], [r16, col[1]])
                ej2 = plsc.load_gather(ejb[b], [r16, col[2]])
                o16 = g * CH + v * 16
                fbig[0, pl.ds(o16, 16)] = ej1 - ei1
                fbig[1, pl.ds(o16, 16)] = ej2 - ei2
                fbig[2, pl.ds(o16, 16)] = ej0 - ei0
                fbig[3, pl.ds(o16, 16)] = phi
            nxt = g + KBUF

            @pl.when(nxt < NCHUNK)
            def _():
                pltpu.async_copy(small_hbm.at[src2d.at[nxt]], eib[b], gi[b])
                pltpu.async_copy(small_hbm.at[dst2d.at[nxt]], ejb[b], gj[b])
        return carry

    lax.fori_loop(0, NCHUNK // KBUF, outer, 0)
    for r in range(4):
        pltpu.sync_copy(fbig.at[r], out_hbm.at[r, pl.ds(wid * EPT, EPT)])


def _stage_c(small, eidx3):
    mesh = plsc.VectorSubcoreMesh(core_axis_name="c", subcore_axis_name="s")
    return pl.kernel(
        _stage_c_body,
        out_type=jax.ShapeDtypeStruct((4, E), jnp.float32),
        mesh=mesh,
        scratch_types=[
            pltpu.VMEM((NCHUNK, CH), jnp.int32),
            pltpu.VMEM((NCHUNK, CH), jnp.int32),
            pltpu.VMEM((4, EPT), jnp.float32),
        ] + [pltpu.VMEM((CH, 8), jnp.float32)] * (2 * KBUF)
          + [pltpu.SemaphoreType.DMA] * (2 * KBUF),
        compiler_params=pltpu.CompilerParams(use_tc_tiling_on_sc=False, needs_layout_passes=False),
    )(small, eidx3)


def _stage_d_kernel(f_ref, w1t_ref, b1_ref, w2t_ref, b2_ref, out_ref):
    f = f_ref[...]
    dx = f[0:1, :]
    dy = f[1:2, :]
    dt = f[2:3, :]
    phi = f[3:4, :]
    dx2 = dx * dx + dy * dy
    ds2 = dx2 - phi * (dt * dt)
    sp = jnp.sqrt(dx2)
    itl = jnp.where(ds2 < 0, 1.0, 0.0).astype(jnp.float32)
    zz = jnp.zeros_like(dx)
    feats = jnp.concatenate([dx, dy, dt, ds2, sp, itl, zz, zz], axis=0)
    h = jnp.maximum(
        jnp.dot(w1t_ref[...], feats, preferred_element_type=jnp.float32)
        + b1_ref[...], 0.0)
    o = jnp.dot(w2t_ref[...], h, preferred_element_type=jnp.float32) + b2_ref[...]
    out_ref[...] = jax.nn.sigmoid(o)


DBLK = 12800


def _stage_d(feat, w1t, b1c, w2t, b2):
    return pl.pallas_call(
        _stage_d_kernel,
        grid=(E // DBLK,),
        in_specs=[
            pl.BlockSpec((4, DBLK), lambda i: (0, i)),
            pl.BlockSpec((32, 8), lambda i: (0, 0)),
            pl.BlockSpec((32, 1), lambda i: (0, 0)),
            pl.BlockSpec((1, 32), lambda i: (0, 0)),
            pl.BlockSpec((1, 1), lambda i: (0, 0)),
        ],
        out_specs=pl.BlockSpec((1, DBLK), lambda i: (0, i)),
        out_shape=jax.ShapeDtypeStruct((1, E), jnp.float32),
    )(feat, w1t, b1c, w2t, b2)


def kernel(x, edge_index, W_gnn, b_gnn, W_phi1, b_phi1, W_phi2, b_phi2,
           W_ep1, b_ep1, W_ep2, b_ep2):
    eidx3 = edge_index.reshape(2, E // CH, CH)
    ones = jnp.ones((N, 1), jnp.float32)
    padz = jnp.zeros((N, AUGH - HALF - 1), jnp.float32)
    xcat = jnp.concatenate(
        [jnp.concatenate([x[:, :HALF], ones, padz], axis=1),
         jnp.concatenate([x[:, HALF:], ones, padz], axis=1)], axis=0)
    zeros = jnp.zeros((RPT, AUGH), jnp.float32)

    partials = _stage_a(xcat, eidx3, zeros)
    small = _stage_b(partials[0], partials[1], W_gnn, b_gnn.reshape(1, D),
                     W_phi1, b_phi1.reshape(1, 64), W_phi2, b_phi2.reshape(1, 1))
    feat = _stage_c(small, eidx3)

    w1t = jnp.concatenate([W_ep1, jnp.zeros((2, 32), jnp.float32)], axis=0).T
    probs = _stage_d(feat, w1t, b_ep1.reshape(32, 1),
                     W_ep2.T, b_ep2.reshape(1, 1))
    return probs.reshape(E)


# speedup vs baseline: 17.7478x; 1.3040x over previous
import jax
import jax.numpy as jnp
from jax import lax
from jax.experimental import pallas as pl
from jax.experimental.pallas import tpu as pltpu
from jax.experimental.pallas import tpu_sc as plsc

N = 10000
D = 128
E = 320000
HALF = 64
NC = 2
NS = 16
NW = NC * NS
EPT = E // NW
EPS = E // NS
CH = 80
NCHUNK = EPT // CH
NCHA = EPS // CH
RPT = 624
TAIL = N - NS * RPT
KBUF = 5


def _stage_a_body(x2_hbm, eidx_hbm, zeros_hbm, out_agg, out_deg,
                  agg_sh, srcf, dstf, deg,
                  r0, r1, r2, r3, r4, g0, g1, g2, g3, g4):
    c = lax.axis_index("c")
    s = lax.axis_index("s")
    rows = [r0, r1, r2, r3, r4]
    gsem = [g0, g1, g2, g3, g4]
    ones16 = jnp.ones((16,), jnp.float32)
    zero16 = jnp.zeros((16,), jnp.float32)

    pltpu.sync_copy(eidx_hbm.at[0, pl.ds(s * EPS, EPS)], srcf)
    pltpu.sync_copy(eidx_hbm.at[1, pl.ds(s * EPS, EPS)], dstf)
    cc = jnp.broadcast_to(c, (16,)).astype(jnp.int32)

    def fix(i, carry):
        v = srcf[pl.ds(i * 16, 16)]
        srcf[pl.ds(i * 16, 16)] = v + v + cc
        return carry

    lax.fori_loop(0, EPS // 16, fix, 0)
    for b in range(KBUF):
        pltpu.async_copy(x2_hbm.at[srcf.at[pl.ds(b * CH, CH)]],
                         rows[b], gsem[b])

    def zloop(i, carry):
        deg[pl.ds(i * 16, 16)] = zero16
        return carry

    lax.fori_loop(0, N // 16, zloop, 0)
    pltpu.sync_copy(zeros_hbm.at[pl.ds(0, RPT)], agg_sh.at[pl.ds(s * RPT, RPT)])

    @pl.when(s == NS - 1)
    def _():
        pltpu.sync_copy(zeros_hbm.at[pl.ds(0, TAIL)],
                        agg_sh.at[pl.ds(NS * RPT, TAIL)])

    plsc.subcore_barrier()

    def outer(o, carry):
        for b in range(KBUF):
            g = o * KBUF + b
            pltpu.make_async_copy(x2_hbm.at[pl.ds(0, CH)], rows[b],
                                  gsem[b]).wait()

            @pl.when(c == 0)
            def _():
                for k in range(CH // 16):
                    dv = dstf[pl.ds(g * CH + k * 16, 16)]
                    plsc.addupdate_scatter(deg, [dv], ones16)

            pltpu.sync_copy(rows[b], agg_sh.at[dstf.at[pl.ds(g * CH, CH)]],
                            add=True)
            nxt = g + KBUF

            @pl.when(nxt < NCHA)
            def _():
                pltpu.async_copy(x2_hbm.at[srcf.at[pl.ds(nxt * CH, CH)]],
                                 rows[b], gsem[b])
        return carry

    lax.fori_loop(0, NCHA // KBUF, outer, 0)
    plsc.subcore_barrier()

    pltpu.sync_copy(agg_sh.at[pl.ds(s * RPT, RPT)],
                    out_agg.at[c, pl.ds(s * RPT, RPT)])

    @pl.when(s == NS - 1)
    def _():
        pltpu.sync_copy(agg_sh.at[pl.ds(NS * RPT, TAIL)],
                        out_agg.at[c, pl.ds(NS * RPT, TAIL)])

    @pl.when(c == 0)
    def _():
        pltpu.sync_copy(deg, out_deg.at[s])


def _stage_a(x2, eidx, zeros):
    mesh = plsc.VectorSubcoreMesh(core_axis_name="c", subcore_axis_name="s")
    return pl.kernel(
        _stage_a_body,
        out_type=[jax.ShapeDtypeStruct((NC, N, HALF), jnp.float32),
                  jax.ShapeDtypeStruct((NS, N), jnp.float32)],
        mesh=mesh,
        scratch_types=[
            pltpu.VMEM_SHARED((N, HALF), jnp.float32),
            pltpu.VMEM((EPS,), jnp.int32),
            pltpu.VMEM((EPS,), jnp.int32),
            pltpu.VMEM((N,), jnp.float32),
        ] + [pltpu.VMEM((CH, HALF), jnp.float32)] * KBUF
          + [pltpu.SemaphoreType.DMA] * KBUF,
        compiler_params=pltpu.CompilerParams(use_tc_tiling_on_sc=False, needs_layout_passes=False),
    )(x2, eidx, zeros)


def _stage_b_kernel(p_ref, dg_ref, wg_ref, bg_ref, w1_ref, b1_ref,
                    w2_ref, b2_ref, out_ref):
    p = p_ref[...]
    deg = jnp.maximum(jnp.sum(dg_ref[...], axis=0), 1.0)
    mean = jnp.concatenate([p[0], p[1]], axis=1) / deg[:, None]
    xe = jnp.maximum(
        jnp.dot(mean, wg_ref[...], preferred_element_type=jnp.float32)
        + bg_ref[...], 0.0)
    h = jnp.maximum(
        jnp.dot(xe, w1_ref[...], preferred_element_type=jnp.float32)
        + b1_ref[...], 0.0)
    phi = jnp.dot(h, w2_ref[...], preferred_element_type=jnp.float32) + b2_ref[...]
    rows = xe.shape[0]
    out_ref[...] = jnp.concatenate(
        [xe[:, 0:3], phi, jnp.zeros((rows, 4), jnp.float32)], axis=1)


def _stage_b(partials, deg, wg, bg, w1, b1, w2, b2):
    return pl.pallas_call(
        _stage_b_kernel,
        grid=(1,),
        in_specs=[
            pl.BlockSpec((NC, N, HALF), lambda i: (0, 0, 0)),
            pl.BlockSpec((NS, N), lambda i: (0, 0)),
            pl.BlockSpec((D, D), lambda i: (0, 0)),
            pl.BlockSpec((1, D), lambda i: (0, 0)),
            pl.BlockSpec((D, 64), lambda i: (0, 0)),
            pl.BlockSpec((1, 64), lambda i: (0, 0)),
            pl.BlockSpec((64, 1), lambda i: (0, 0)),
            pl.BlockSpec((1, 1), lambda i: (0, 0)),
        ],
        out_specs=pl.BlockSpec((N, 8), lambda i: (0, 0)),
        out_shape=jax.ShapeDtypeStruct((N, 8), jnp.float32),
    )(partials, deg, wg, bg, w1, b1, w2, b2)


def _stage_c_body(small_hbm, eidx_hbm, out_hbm,
                  srcf, dstf, fbig,
                  ei0b, ei1b, ei2b, ei3b, ei4b,
                  ej0b, ej1b, ej2b, ej3b, ej4b,
                  gi0, gi1, gi2, gi3, gi4,
                  gj0, gj1, gj2, gj3, gj4):
    c = lax.axis_index("c")
    s = lax.axis_index("s")
    wid = c * NS + s
    eib = [ei0b, ei1b, ei2b, ei3b, ei4b]
    ejb = [ej0b, ej1b, ej2b, ej3b, ej4b]
    gi = [gi0, gi1, gi2, gi3, gi4]
    gj = [gj0, gj1, gj2, gj3, gj4]
    col = [jnp.full((16,), k, jnp.int32) for k in range(4)]

    pltpu.sync_copy(eidx_hbm.at[0, pl.ds(wid * EPT, EPT)], srcf)
    pltpu.sync_copy(eidx_hbm.at[1, pl.ds(wid * EPT, EPT)], dstf)
    for b in range(KBUF):
        pltpu.async_copy(small_hbm.at[srcf.at[pl.ds(b * CH, CH)]],
                         eib[b], gi[b])
        pltpu.async_copy(small_hbm.at[dstf.at[pl.ds(b * CH, CH)]],
                         ejb[b], gj[b])

    def outer(o, carry):
        for b in range(KBUF):
            g = o * KBUF + b
            pltpu.make_async_copy(small_hbm.at[pl.ds(0, CH)], eib[b],
                                  gi[b]).wait()
            pltpu.make_async_copy(small_hbm.at[pl.ds(0, CH)], ejb[b],
                                  gj[b]).wait()
            for v in range(CH // 16):
                r16 = lax.iota(jnp.int32, 16) + v * 16
                ei0 = plsc.load_gather(eib[b], [r16, col[0]])
                ei1 = plsc.load_gather(eib[b], [r16, col[1]])
                ei2 = plsc.load_gather(eib[b], [r16, col[2]])
                phi = plsc.load_gather(eib[b], [r16, col[3]])
                ej0 = plsc.load_gather(ejb[b], [r16, col[0]])
                ej1 = plsc.load_gather(ejb[b], [r16, col[1]])
                ej2 = plsc.load_gather(ejb[b], [r16, col[2]])
                o16 = g * CH + v * 16
                fbig[0, pl.ds(o16, 16)] = ej1 - ei1
                fbig[1, pl.ds(o16, 16)] = ej2 - ei2
                fbig[2, pl.ds(o16, 16)] = ej0 - ei0
                fbig[3, pl.ds(o16, 16)] = phi
            nxt = g + KBUF

            @pl.when(nxt < NCHUNK)
            def _():
                pltpu.async_copy(small_hbm.at[srcf.at[pl.ds(nxt * CH, CH)]],
                                 eib[b], gi[b])
                pltpu.async_copy(small_hbm.at[dstf.at[pl.ds(nxt * CH, CH)]],
                                 ejb[b], gj[b])
        return carry

    lax.fori_loop(0, NCHUNK // KBUF, outer, 0)
    for r in range(4):
        pltpu.sync_copy(fbig.at[r], out_hbm.at[r, pl.ds(wid * EPT, EPT)])


def _stage_c(small, eidx):
    mesh = plsc.VectorSubcoreMesh(core_axis_name="c", subcore_axis_name="s")
    return pl.kernel(
        _stage_c_body,
        out_type=jax.ShapeDtypeStruct((4, E), jnp.float32),
        mesh=mesh,
        scratch_types=[
            pltpu.VMEM((EPT,), jnp.int32),
            pltpu.VMEM((EPT,), jnp.int32),
            pltpu.VMEM((4, EPT), jnp.float32),
        ] + [pltpu.VMEM((CH, 8), jnp.float32)] * (2 * KBUF)
          + [pltpu.SemaphoreType.DMA] * (2 * KBUF),
        compiler_params=pltpu.CompilerParams(use_tc_tiling_on_sc=False, needs_layout_passes=False),
    )(small, eidx)


def _stage_d_kernel(f_ref, w1t_ref, b1_ref, w2t_ref, b2_ref, out_ref):
    f = f_ref[...]
    dx = f[0:1, :]
    dy = f[1:2, :]
    dt = f[2:3, :]
    phi = f[3:4, :]
    dx2 = dx * dx + dy * dy
    ds2 = dx2 - phi * (dt * dt)
    sp = jnp.sqrt(dx2)
    itl = jnp.where(ds2 < 0, 1.0, 0.0).astype(jnp.float32)
    zz = jnp.zeros_like(dx)
    feats = jnp.concatenate([dx, dy, dt, ds2, sp, itl, zz, zz], axis=0)
    h = jnp.maximum(
        jnp.dot(w1t_ref[...], feats, preferred_element_type=jnp.float32)
        + b1_ref[...], 0.0)
    o = jnp.dot(w2t_ref[...], h, preferred_element_type=jnp.float32) + b2_ref[...]
    out_ref[...] = jax.nn.sigmoid(o)


DBLK = 32000


def _stage_d(feat, w1t, b1c, w2t, b2):
    return pl.pallas_call(
        _stage_d_kernel,
        grid=(E // DBLK,),
        in_specs=[
            pl.BlockSpec((4, DBLK), lambda i: (0, i)),
            pl.BlockSpec((32, 8), lambda i: (0, 0)),
            pl.BlockSpec((32, 1), lambda i: (0, 0)),
            pl.BlockSpec((1, 32), lambda i: (0, 0)),
            pl.BlockSpec((1, 1), lambda i: (0, 0)),
        ],
        out_specs=pl.BlockSpec((1, DBLK), lambda i: (0, i)),
        out_shape=jax.ShapeDtypeStruct((1, E), jnp.float32),
    )(feat, w1t, b1c, w2t, b2)


def kernel(x, edge_index, W_gnn, b_gnn, W_phi1, b_phi1, W_phi2, b_phi2,
           W_ep1, b_ep1, W_ep2, b_ep2):
    x2 = x.reshape(2 * N, HALF)
    zeros = jnp.zeros((RPT, HALF), jnp.float32)

    agg, deg = _stage_a(x2, edge_index, zeros)
    small = _stage_b(agg, deg, W_gnn, b_gnn.reshape(1, D),
                     W_phi1, b_phi1.reshape(1, 64), W_phi2, b_phi2.reshape(1, 1))
    feat = _stage_c(small, edge_index)

    w1t = jnp.concatenate([W_ep1, jnp.zeros((2, 32), jnp.float32)], axis=0).T
    probs = _stage_d(feat, w1t, b_ep1.reshape(32, 1),
                     W_ep2.T, b_ep2.reshape(1, 1))
    return probs.reshape(E)


# speedup vs baseline: 18.7104x; 1.0542x over previous
import jax
import jax.numpy as jnp
from jax import lax
from jax.experimental import pallas as pl
from jax.experimental.pallas import tpu as pltpu
from jax.experimental.pallas import tpu_sc as plsc

N = 10000
D = 128
E = 320000
HALF = 64
NC = 2
NS = 16
NW = NC * NS
EPT = E // NW
EPS = E // NS
CH = 80
NCHUNK = EPT // CH
NCHA = EPS // CH
RPT = 624
TAIL = N - NS * RPT
KBUF = 5


def _stage_a_body(x2_hbm, eidx_hbm, zeros_hbm, out_agg, out_deg,
                  agg_sh, srcf, dstf, deg,
                  r0, r1, r2, r3, r4, g0, g1, g2, g3, g4):
    c = lax.axis_index("c")
    s = lax.axis_index("s")
    rows = [r0, r1, r2, r3, r4]
    gsem = [g0, g1, g2, g3, g4]
    ones16 = jnp.ones((16,), jnp.float32)
    zero16 = jnp.zeros((16,), jnp.float32)

    pltpu.sync_copy(eidx_hbm.at[0, pl.ds(s * EPS, EPS)], srcf)
    pltpu.sync_copy(eidx_hbm.at[1, pl.ds(s * EPS, EPS)], dstf)
    cc = jnp.broadcast_to(c, (16,)).astype(jnp.int32)

    def fix(i, carry):
        v = srcf[pl.ds(i * 16, 16)]
        srcf[pl.ds(i * 16, 16)] = v + v + cc
        return carry

    lax.fori_loop(0, EPS // 16, fix, 0)
    for b in range(KBUF):
        pltpu.async_copy(x2_hbm.at[srcf.at[pl.ds(b * CH, CH)]],
                         rows[b], gsem[b])

    def zloop(i, carry):
        deg[pl.ds(i * 16, 16)] = zero16
        return carry

    lax.fori_loop(0, N // 16, zloop, 0)
    pltpu.sync_copy(zeros_hbm.at[pl.ds(0, RPT)], agg_sh.at[pl.ds(s * RPT, RPT)])

    @pl.when(s == NS - 1)
    def _():
        pltpu.sync_copy(zeros_hbm.at[pl.ds(0, TAIL)],
                        agg_sh.at[pl.ds(NS * RPT, TAIL)])

    plsc.subcore_barrier()

    def outer(o, carry):
        for b in range(KBUF):
            g = o * KBUF + b
            pltpu.make_async_copy(x2_hbm.at[pl.ds(0, CH)], rows[b],
                                  gsem[b]).wait()

            @pl.when(c == 0)
            def _():
                for k in range(CH // 16):
                    dv = dstf[pl.ds(g * CH + k * 16, 16)]
                    plsc.addupdate_scatter(deg, [dv], ones16)

            pltpu.sync_copy(rows[b], agg_sh.at[dstf.at[pl.ds(g * CH, CH)]],
                            add=True)
            nxt = g + KBUF

            @pl.when(nxt < NCHA)
            def _():
                pltpu.async_copy(x2_hbm.at[srcf.at[pl.ds(nxt * CH, CH)]],
                                 rows[b], gsem[b])
        return carry

    lax.fori_loop(0, NCHA // KBUF, outer, 0)
    plsc.subcore_barrier()

    pltpu.sync_copy(agg_sh.at[pl.ds(s * RPT, RPT)],
                    out_agg.at[pl.ds(s * RPT, RPT), pl.ds(c * HALF, HALF)])

    @pl.when(s == NS - 1)
    def _():
        pltpu.sync_copy(agg_sh.at[pl.ds(NS * RPT, TAIL)],
                        out_agg.at[pl.ds(NS * RPT, TAIL), pl.ds(c * HALF, HALF)])

    @pl.when(c == 0)
    def _():
        pltpu.sync_copy(deg, out_deg.at[s])


def _stage_a(x2, eidx, zeros):
    mesh = plsc.VectorSubcoreMesh(core_axis_name="c", subcore_axis_name="s")
    return pl.kernel(
        _stage_a_body,
        out_type=[jax.ShapeDtypeStruct((N, D), jnp.float32),
                  jax.ShapeDtypeStruct((NS, N), jnp.float32)],
        mesh=mesh,
        scratch_types=[
            pltpu.VMEM_SHARED((N, HALF), jnp.float32),
            pltpu.VMEM((EPS,), jnp.int32),
            pltpu.VMEM((EPS,), jnp.int32),
            pltpu.VMEM((N,), jnp.float32),
        ] + [pltpu.VMEM((CH, HALF), jnp.float32)] * KBUF
          + [pltpu.SemaphoreType.DMA] * KBUF,
        compiler_params=pltpu.CompilerParams(use_tc_tiling_on_sc=False, needs_layout_passes=False),
    )(x2, eidx, zeros)


def _stage_b_kernel(p_ref, dg_ref, wg_ref, bg_ref, w1_ref, b1_ref,
                    w2_ref, b2_ref, out_ref):
    deg = jnp.maximum(jnp.sum(dg_ref[...], axis=0), 1.0)
    mean = p_ref[...] / deg[:, None]
    xe = jnp.maximum(
        jnp.dot(mean, wg_ref[...], preferred_element_type=jnp.float32)
        + bg_ref[...], 0.0)
    h = jnp.maximum(
        jnp.dot(xe, w1_ref[...], preferred_element_type=jnp.float32)
        + b1_ref[...], 0.0)
    phi = jnp.dot(h, w2_ref[...], preferred_element_type=jnp.float32) + b2_ref[...]
    rows = xe.shape[0]
    out_ref[...] = jnp.concatenate(
        [xe[:, 0:3], phi, jnp.zeros((rows, 4), jnp.float32)], axis=1)


def _stage_b(partials, deg, wg, bg, w1, b1, w2, b2):
    return pl.pallas_call(
        _stage_b_kernel,
        grid=(1,),
        in_specs=[
            pl.BlockSpec((N, D), lambda i: (0, 0)),
            pl.BlockSpec((NS, N), lambda i: (0, 0)),
            pl.BlockSpec((D, D), lambda i: (0, 0)),
            pl.BlockSpec((1, D), lambda i: (0, 0)),
            pl.BlockSpec((D, 64), lambda i: (0, 0)),
            pl.BlockSpec((1, 64), lambda i: (0, 0)),
            pl.BlockSpec((64, 1), lambda i: (0, 0)),
            pl.BlockSpec((1, 1), lambda i: (0, 0)),
        ],
        out_specs=pl.BlockSpec((N, 8), lambda i: (0, 0)),
        out_shape=jax.ShapeDtypeStruct((N, 8), jnp.float32),
    )(partials, deg, wg, bg, w1, b1, w2, b2)


def _stage_c_body(small_hbm, eidx_hbm, out_hbm,
                  srcf, dstf, fbig,
                  ei0b, ei1b, ei2b, ei3b, ei4b,
                  ej0b, ej1b, ej2b, ej3b, ej4b,
                  gi0, gi1, gi2, gi3, gi4,
                  gj0, gj1, gj2, gj3, gj4):
    c = lax.axis_index("c")
    s = lax.axis_index("s")
    wid = c * NS + s
    eib = [ei0b, ei1b, ei2b, ei3b, ei4b]
    ejb = [ej0b, ej1b, ej2b, ej3b, ej4b]
    gi = [gi0, gi1, gi2, gi3, gi4]
    gj = [gj0, gj1, gj2, gj3, gj4]
    col = [jnp.full((16,), k, jnp.int32) for k in range(4)]

    pltpu.sync_copy(eidx_hbm.at[0, pl.ds(wid * EPT, EPT)], srcf)
    pltpu.sync_copy(eidx_hbm.at[1, pl.ds(wid * EPT, EPT)], dstf)
    for b in range(KBUF):
        pltpu.async_copy(small_hbm.at[srcf.at[pl.ds(b * CH, CH)]],
                         eib[b], gi[b])
        pltpu.async_copy(small_hbm.at[dstf.at[pl.ds(b * CH, CH)]],
                         ejb[b], gj[b])

    def outer(o, carry):
        for b in range(KBUF):
            g = o * KBUF + b
            pltpu.make_async_copy(small_hbm.at[pl.ds(0, CH)], eib[b],
                                  gi[b]).wait()
            pltpu.make_async_copy(small_hbm.at[pl.ds(0, CH)], ejb[b],
                                  gj[b]).wait()
            for v in range(CH // 16):
                r16 = lax.iota(jnp.int32, 16) + v * 16
                ei0 = plsc.load_gather(eib[b], [r16, col[0]])
                ei1 = plsc.load_gather(eib[b], [r16, col[1]])
                ei2 = plsc.load_gather(eib[b], [r16, col[2]])
                phi = plsc.load_gather(eib[b], [r16, col[3]])
                ej0 = plsc.load_gather(ejb[b], [r16, col[0]])
                ej1 = plsc.load_gather(ejb[b], [r16, col[1]])
                ej2 = plsc.load_gather(ejb[b], [r16, col[2]])
                o16 = g * CH + v * 16
                fbig[0, pl.ds(o16, 16)] = ej1 - ei1
                fbig[1, pl.ds(o16, 16)] = ej2 - ei2
                fbig[2, pl.ds(o16, 16)] = ej0 - ei0
                fbig[3, pl.ds(o16, 16)] = phi
            nxt = g + KBUF

            @pl.when(nxt < NCHUNK)
            def _():
                pltpu.async_copy(small_hbm.at[srcf.at[pl.ds(nxt * CH, CH)]],
                                 eib[b], gi[b])
                pltpu.async_copy(small_hbm.at[dstf.at[pl.ds(nxt * CH, CH)]],
                                 ejb[b], gj[b])
        return carry

    lax.fori_loop(0, NCHUNK // KBUF, outer, 0)
    for r in range(4):
        pltpu.sync_copy(fbig.at[r], out_hbm.at[r, pl.ds(wid * EPT, EPT)])


def _stage_c(small, eidx):
    mesh = plsc.VectorSubcoreMesh(core_axis_name="c", subcore_axis_name="s")
    return pl.kernel(
        _stage_c_body,
        out_type=jax.ShapeDtypeStruct((4, E), jnp.float32),
        mesh=mesh,
        scratch_types=[
            pltpu.VMEM((EPT,), jnp.int32),
            pltpu.VMEM((EPT,), jnp.int32),
            pltpu.VMEM((4, EPT), jnp.float32),
        ] + [pltpu.VMEM((CH, 8), jnp.float32)] * (2 * KBUF)
          + [pltpu.SemaphoreType.DMA] * (2 * KBUF),
        compiler_params=pltpu.CompilerParams(use_tc_tiling_on_sc=False, needs_layout_passes=False),
    )(small, eidx)


def _stage_d_kernel(f_ref, w1t_ref, b1_ref, w2t_ref, b2_ref, out_ref):
    f = f_ref[...]
    dx = f[0:1, :]
    dy = f[1:2, :]
    dt = f[2:3, :]
    phi = f[3:4, :]
    dx2 = dx * dx + dy * dy
    ds2 = dx2 - phi * (dt * dt)
    sp = jnp.sqrt(dx2)
    itl = jnp.where(ds2 < 0, 1.0, 0.0).astype(jnp.float32)
    zz = jnp.zeros_like(dx)
    feats = jnp.concatenate([dx, dy, dt, ds2, sp, itl, zz, zz], axis=0)
    h = jnp.maximum(
        jnp.dot(w1t_ref[...], feats, preferred_element_type=jnp.float32)
        + b1_ref[...], 0.0)
    o = jnp.dot(w2t_ref[...], h, preferred_element_type=jnp.float32) + b2_ref[...]
    out_ref[...] = jax.nn.sigmoid(o)


DBLK = 32000


def _stage_d(feat, w1t, b1c, w2t, b2):
    return pl.pallas_call(
        _stage_d_kernel,
        grid=(E // DBLK,),
        in_specs=[
            pl.BlockSpec((4, DBLK), lambda i: (0, i)),
            pl.BlockSpec((32, 8), lambda i: (0, 0)),
            pl.BlockSpec((32, 1), lambda i: (0, 0)),
            pl.BlockSpec((1, 32), lambda i: (0, 0)),
            pl.BlockSpec((1, 1), lambda i: (0, 0)),
        ],
        out_specs=pl.BlockSpec((1, DBLK), lambda i: (0, i)),
        out_shape=jax.ShapeDtypeStruct((1, E), jnp.float32),
    )(feat, w1t, b1c, w2t, b2)


def kernel(x, edge_index, W_gnn, b_gnn, W_phi1, b_phi1, W_phi2, b_phi2,
           W_ep1, b_ep1, W_ep2, b_ep2):
    x2 = x.reshape(2 * N, HALF)
    zeros = jnp.zeros((RPT, HALF), jnp.float32)

    agg, deg = _stage_a(x2, edge_index, zeros)
    small = _stage_b(agg, deg, W_gnn, b_gnn.reshape(1, D),
                     W_phi1, b_phi1.reshape(1, 64), W_phi2,
                     b_phi2.reshape(1, 1))
    feat = _stage_c(small, edge_index)

    w1t = jnp.concatenate([W_ep1, jnp.zeros((2, 32), jnp.float32)], axis=0).T
    probs = _stage_d(feat, w1t, b_ep1.reshape(32, 1),
                     W_ep2.T, b_ep2.reshape(1, 1))
    return probs.reshape(E)
